# Initial kernel scaffold; baseline (speedup 1.0000x reference)
import functools

import jax
import jax.numpy as jnp
from jax import lax
from jax.experimental import pallas as pl
from jax.experimental.pallas import tpu as pltpu
from jax.experimental.pallas import tpu_sc as plsc

N = 10000
E = 320000
R = 8
D = 128
D_OUT = 16
NUM_REPOS = 5000
NSEG = N * R

NC = 2
NS = 16
L = 16

CH = 80
EPT_CNT = E // NS
EPT_AGG = E // (NC * NS)
NCH_CNT = EPT_CNT // CH
NCH_AGG = EPT_AGG // CH
SEG_PT = NSEG // NS
ROWS_PT = N // NS

_SC_MESH = plsc.VectorSubcoreMesh(
    core_axis_name="c", subcore_axis_name="s", num_cores=NC, num_subcores=NS)


@functools.partial(
    pl.kernel,
    out_type=jax.ShapeDtypeStruct((2, NSEG), jnp.float32),
    mesh=_SC_MESH,
    scratch_types=[
        pltpu.VMEM((EPT_CNT,), jnp.int32),
        pltpu.VMEM((EPT_CNT,), jnp.int32),
        pltpu.VMEM((CH,), jnp.int32),
        pltpu.VMEM((CH,), jnp.float32),
        pltpu.VMEM_SHARED((NSEG,), jnp.float32),
    ],
)
def _sc_counts(dst_all, et_all, zseg, out, dst_v, et_v, seg_v, ones_v, cnt_sh):
    c = lax.axis_index("c")
    s = lax.axis_index("s")
    for g in range(CH // L):
        ones_v[pl.ds(g * L, L)] = jnp.ones((L,), jnp.float32)
    zoff = s * SEG_PT
    pltpu.sync_copy(zseg.at[pl.ds(zoff, SEG_PT)], cnt_sh.at[pl.ds(zoff, SEG_PT)])
    base = s * EPT_CNT
    pltpu.sync_copy(dst_all.at[c, pl.ds(base, EPT_CNT)], dst_v)
    pltpu.sync_copy(et_all.at[c, pl.ds(base, EPT_CNT)], et_v)
    plsc.subcore_barrier()

    def chunk(i, carry):
        off = i * CH
        for g in range(CH // L):
            d16 = dst_v[pl.ds(off + g * L, L)]
            e16 = et_v[pl.ds(off + g * L, L)]
            seg_v[pl.ds(g * L, L)] = d16 * R + e16
        pltpu.sync_copy(ones_v, cnt_sh.at[seg_v], add=True)
        return carry

    lax.fori_loop(0, NCH_CNT, chunk, 0)
    plsc.subcore_barrier()
    pltpu.sync_copy(cnt_sh.at[pl.ds(zoff, SEG_PT)], out.at[c, pl.ds(zoff, SEG_PT)])


@functools.partial(
    pl.kernel,
    out_type=jax.ShapeDtypeStruct((NC, N, D), jnp.float32),
    mesh=_SC_MESH,
    scratch_types=[
        pltpu.VMEM((EPT_AGG,), jnp.int32),
        pltpu.VMEM((EPT_AGG,), jnp.int32),
        pltpu.VMEM((EPT_AGG,), jnp.int32),
        pltpu.VMEM((CH,), jnp.int32),
        pltpu.VMEM((CH,), jnp.int32),
        pltpu.VMEM((CH,), jnp.int32),
        pltpu.VMEM((CH,), jnp.float32),
        pltpu.VMEM((CH, D), jnp.float32),
        pltpu.VMEM_SHARED((N, D), jnp.float32),
        pltpu.SemaphoreType.DMA,
        pltpu.SemaphoreType.DMA,
    ],
)
def _sc_agg(hw, src_h, dst_h, et_h, w_h, zrow, out,
            src_v, et_v, dst_v, gidx_v, seg_v, dch_v, wch_v, row_v,
            acc_sh, gsem, wsem):
    c = lax.axis_index("c")
    s = lax.axis_index("s")
    wid = c * NS + s
    base = wid * EPT_AGG
    roff = s * ROWS_PT
    pltpu.sync_copy(zrow.at[pl.ds(roff, ROWS_PT)], acc_sh.at[pl.ds(roff, ROWS_PT)])
    pltpu.sync_copy(src_h.at[pl.ds(base, EPT_AGG)], src_v)
    pltpu.sync_copy(et_h.at[pl.ds(base, EPT_AGG)], et_v)
    pltpu.sync_copy(dst_h.at[pl.ds(base, EPT_AGG)], dst_v)
    plsc.subcore_barrier()

    def chunk(i, carry):
        off = i * CH
        for g in range(CH // L):
            s16 = src_v[pl.ds(off + g * L, L)]
            e16 = et_v[pl.ds(off + g * L, L)]
            d16 = dst_v[pl.ds(off + g * L, L)]
            gidx_v[pl.ds(g * L, L)] = e16 * N + s16
            seg_v[pl.ds(g * L, L)] = d16 * R + e16
            dch_v[pl.ds(g * L, L)] = d16
        cp_r = pltpu.async_copy(hw.at[gidx_v], row_v, gsem)
        cp_w = pltpu.async_copy(w_h.at[seg_v], wch_v, wsem)
        cp_r.wait()
        cp_w.wait()

        def edge(e, inner):
            wv = plsc.load_gather(wch_v, [jnp.full((L,), e, jnp.int32)])
            for j in range(D // L):
                row_v[e, pl.ds(j * L, L)] = row_v[e, pl.ds(j * L, L)] * wv
            return inner

        lax.fori_loop(0, CH, edge, 0)
        pltpu.sync_copy(row_v, acc_sh.at[dch_v], add=True)
        return carry

    lax.fori_loop(0, NCH_AGG, chunk, 0)
    plsc.subcore_barrier()
    pltpu.sync_copy(acc_sh.at[pl.ds(roff, ROWS_PT)],
                    out.at[c, pl.ds(roff, ROWS_PT)])


def _weights_kernel(cnt):
    def body(cnt_ref, w_ref):
        w_ref[...] = 1.0 / jnp.maximum(cnt_ref[...], 1.0)

    w = pl.pallas_call(
        body,
        out_shape=jax.ShapeDtypeStruct((2, NSEG // D, D), jnp.float32),
    )(cnt.reshape(2, NSEG // D, D))
    return w.reshape(2, NSEG)


_BN = 1000


def _dense1_kernel(x, Wm, bm, W1, root1, b1):
    def body(x_ref, wm_ref, bm_ref, w1_ref, rt_ref, b1_ref, hw_ref, hr_ref):
        h = jnp.maximum(
            jnp.dot(x_ref[...], wm_ref[...], preferred_element_type=jnp.float32)
            + bm_ref[...], 0.0)
        for r in range(R):
            hw_ref[r] = jnp.dot(h, w1_ref[r], preferred_element_type=jnp.float32)
        hr_ref[...] = jnp.dot(h, rt_ref[...],
                              preferred_element_type=jnp.float32) + b1_ref[...]

    return pl.pallas_call(
        body,
        grid=(N // _BN,),
        in_specs=[
            pl.BlockSpec((_BN, D), lambda i: (i, 0)),
            pl.BlockSpec((D, D), lambda i: (0, 0)),
            pl.BlockSpec((1, D), lambda i: (0, 0)),
            pl.BlockSpec((R, D, D), lambda i: (0, 0, 0)),
            pl.BlockSpec((D, D), lambda i: (0, 0)),
            pl.BlockSpec((1, D), lambda i: (0, 0)),
        ],
        out_specs=[
            pl.BlockSpec((R, _BN, D), lambda i: (0, i, 0)),
            pl.BlockSpec((_BN, D), lambda i: (i, 0)),
        ],
        out_shape=[
            jax.ShapeDtypeStruct((R, N, D), jnp.float32),
            jax.ShapeDtypeStruct((N, D), jnp.float32),
        ],
    )(x, Wm, bm.reshape(1, D), W1, root1, b1.reshape(1, D))


def _dense2_kernel(acc, hr_prev, W, root, b):
    def body(acc_ref, hp_ref, w_ref, rt_ref, b_ref, hw_ref, hr_ref):
        h = jnp.maximum(acc_ref[0] + acc_ref[1] + hp_ref[...], 0.0)
        for r in range(R):
            hw_ref[r] = jnp.dot(h, w_ref[r], preferred_element_type=jnp.float32)
        hr_ref[...] = jnp.dot(h, rt_ref[...],
                              preferred_element_type=jnp.float32) + b_ref[...]

    return pl.pallas_call(
        body,
        grid=(N // _BN,),
        in_specs=[
            pl.BlockSpec((NC, _BN, D), lambda i: (0, i, 0)),
            pl.BlockSpec((_BN, D), lambda i: (i, 0)),
            pl.BlockSpec((R, D, D), lambda i: (0, 0, 0)),
            pl.BlockSpec((D, D), lambda i: (0, 0)),
            pl.BlockSpec((1, D), lambda i: (0, 0)),
        ],
        out_specs=[
            pl.BlockSpec((R, _BN, D), lambda i: (0, i, 0)),
            pl.BlockSpec((_BN, D), lambda i: (i, 0)),
        ],
        out_shape=[
            jax.ShapeDtypeStruct((R, N, D), jnp.float32),
            jax.ShapeDtypeStruct((N, D), jnp.float32),
        ],
    )(acc, hr_prev, W, root, b.reshape(1, D))


def _final_kernel(acc, hr_prev, Wc, bc):
    def body(acc_ref, hp_ref, wc_ref, bc_ref, out_ref):
        h = jnp.maximum(acc_ref[0] + acc_ref[1] + hp_ref[...], 0.0)
        out_ref[...] = jnp.dot(h, wc_ref[...],
                               preferred_element_type=jnp.float32) + bc_ref[...]

    return pl.pallas_call(
        body,
        grid=(NUM_REPOS // _BN,),
        in_specs=[
            pl.BlockSpec((NC, _BN, D), lambda i: (0, i, 0)),
            pl.BlockSpec((_BN, D), lambda i: (i, 0)),
            pl.BlockSpec((D, D_OUT), lambda i: (0, 0)),
            pl.BlockSpec((1, D_OUT), lambda i: (0, 0)),
        ],
        out_specs=pl.BlockSpec((_BN, D_OUT), lambda i: (i, 0)),
        out_shape=jax.ShapeDtypeStruct((NUM_REPOS, D_OUT), jnp.float32),
    )(acc, hr_prev, Wc, bc.reshape(1, D_OUT))


def kernel(x, edge_index_ru, edge_index_ur, edge_type_ru, edge_type_ur,
           Wm, bm, W1, root1, b1, W2, root2, b2, Wc, bc):
    src1, dst1 = edge_index_ru[0], edge_index_ru[1]
    src2, dst2 = edge_index_ur[0], edge_index_ur[1]
    dst_all = jnp.stack([dst1, dst2])
    et_all = jnp.stack([edge_type_ru, edge_type_ur])
    zseg = jnp.zeros((NSEG,), jnp.float32)
    zrow = jnp.zeros((N, D), jnp.float32)

    cnt = _sc_counts(dst_all, et_all, zseg)
    w = _weights_kernel(cnt)

    hw1, hr1 = _dense1_kernel(x, Wm, bm, W1, root1, b1)
    acc1 = _sc_agg(hw1.reshape(R * N, D), src1, dst1, edge_type_ru, w[0], zrow)
    hw2, hr2 = _dense2_kernel(acc1, hr1, W2, root2, b2)
    acc2 = _sc_agg(hw2.reshape(R * N, D), src2, dst2, edge_type_ur, w[1], zrow)
    return _final_kernel(acc2, hr2, Wc, bc)


# speedup vs baseline: 21.0684x; 21.0684x over previous
import functools

import jax
import jax.numpy as jnp
from jax import lax
from jax.experimental import pallas as pl
from jax.experimental.pallas import tpu as pltpu
from jax.experimental.pallas import tpu_sc as plsc

N = 10000
E = 320000
R = 8
D = 128
D_OUT = 16
NUM_REPOS = 5000
NSEG = N * R

NC = 2
NS = 16
L = 16

CH = 80
EPT_CNT = E // NS
EPT_AGG = E // (NC * NS)
NCH_CNT = EPT_CNT // CH
NCH_AGG = EPT_AGG // CH
SEG_PT = NSEG // NS
ROWS_PT = 624
ROWS_TAIL = N - ROWS_PT * NS

_SC_MESH = plsc.VectorSubcoreMesh(
    core_axis_name="c", subcore_axis_name="s", num_cores=NC, num_subcores=NS)


@functools.partial(
    pl.kernel,
    out_type=jax.ShapeDtypeStruct((2 * NSEG,), jnp.float32),
    mesh=_SC_MESH,
    scratch_types=[
        pltpu.VMEM((EPT_CNT,), jnp.int32),
        pltpu.VMEM((EPT_CNT,), jnp.int32),
        pltpu.VMEM((CH,), jnp.int32),
        pltpu.VMEM((CH,), jnp.float32),
        pltpu.VMEM((SEG_PT,), jnp.float32),
        pltpu.VMEM_SHARED((NSEG,), jnp.float32),
    ],
)
def _sc_counts(dst_cat, et_cat, out, dst_v, et_v, seg_v, ones_v, zb_v, cnt_sh):
    c = lax.axis_index("c")
    s = lax.axis_index("s")
    for g in range(CH // L):
        ones_v[pl.ds(g * L, L)] = jnp.ones((L,), jnp.float32)

    def zfill(i, carry):
        zb_v[pl.ds(i * L, L)] = jnp.zeros((L,), jnp.float32)
        return carry

    lax.fori_loop(0, SEG_PT // L - 1, zfill, 0)
    zb_v[pl.ds(SEG_PT - L, L)] = jnp.zeros((L,), jnp.float32)
    zoff = pl.multiple_of(s * SEG_PT, 8)
    pltpu.sync_copy(zb_v, cnt_sh.at[pl.ds(zoff, SEG_PT)])
    base = pl.multiple_of(c * E + s * EPT_CNT, 8)
    pltpu.sync_copy(dst_cat.at[pl.ds(base, EPT_CNT)], dst_v)
    pltpu.sync_copy(et_cat.at[pl.ds(base, EPT_CNT)], et_v)
    plsc.subcore_barrier()

    def chunk(i, carry):
        off = i * CH
        for g in range(CH // L):
            d16 = dst_v[pl.ds(off + g * L, L)]
            e16 = et_v[pl.ds(off + g * L, L)]
            seg_v[pl.ds(g * L, L)] = d16 * R + e16
        pltpu.sync_copy(ones_v, cnt_sh.at[seg_v], add=True)
        return carry

    lax.fori_loop(0, NCH_CNT, chunk, 0)
    plsc.subcore_barrier()
    pltpu.sync_copy(cnt_sh.at[pl.ds(zoff, SEG_PT)], zb_v)
    ooff = pl.multiple_of(c * NSEG + zoff, 8)
    pltpu.sync_copy(zb_v, out.at[pl.ds(ooff, SEG_PT)])


@functools.partial(
    pl.kernel,
    out_type=jax.ShapeDtypeStruct((NC, N, D), jnp.float32),
    mesh=_SC_MESH,
    scratch_types=[
        pltpu.VMEM((EPT_AGG,), jnp.int32),
        pltpu.VMEM((EPT_AGG,), jnp.int32),
        pltpu.VMEM((EPT_AGG,), jnp.int32),
        pltpu.VMEM((CH,), jnp.int32),
        pltpu.VMEM((CH,), jnp.int32),
        pltpu.VMEM((CH,), jnp.int32),
        pltpu.VMEM((CH,), jnp.float32),
        pltpu.VMEM((CH, D), jnp.float32),
        pltpu.VMEM_SHARED((N, D), jnp.float32),
        pltpu.SemaphoreType.DMA,
        pltpu.SemaphoreType.DMA,
    ],
)
def _sc_agg(hw, src_h, dst_h, et_h, w_h, out,
            src_v, et_v, dst_v, gidx_v, seg_v, dch_v, wch_v, row_v,
            acc_sh, gsem, wsem):
    c = lax.axis_index("c")
    s = lax.axis_index("s")
    base = pl.multiple_of((c * NS + s) * EPT_AGG, 8)
    roff = pl.multiple_of(s * ROWS_PT, 8)
    toff = pl.multiple_of(ROWS_PT * NS, 8)

    def zfill(i, carry):
        for j in range(D // L):
            row_v[i, pl.ds(j * L, L)] = jnp.zeros((L,), jnp.float32)
        return carry

    lax.fori_loop(0, CH, zfill, 0)
    for k in range(ROWS_PT // CH):
        pltpu.sync_copy(row_v, acc_sh.at[pl.ds(roff + k * CH, CH)])
    _rem = ROWS_PT - (ROWS_PT // CH) * CH
    pltpu.sync_copy(row_v.at[pl.ds(0, _rem)],
                    acc_sh.at[pl.ds(roff + ROWS_PT - _rem, _rem)])

    @pl.when(s == NS - 1)
    def _zero_tail():
        pltpu.sync_copy(row_v.at[pl.ds(0, ROWS_TAIL)],
                        acc_sh.at[pl.ds(toff, ROWS_TAIL)])

    pltpu.sync_copy(src_h.at[pl.ds(base, EPT_AGG)], src_v)
    pltpu.sync_copy(et_h.at[pl.ds(base, EPT_AGG)], et_v)
    pltpu.sync_copy(dst_h.at[pl.ds(base, EPT_AGG)], dst_v)
    plsc.subcore_barrier()

    def chunk(i, carry):
        off = i * CH
        for g in range(CH // L):
            s16 = src_v[pl.ds(off + g * L, L)]
            e16 = et_v[pl.ds(off + g * L, L)]
            d16 = dst_v[pl.ds(off + g * L, L)]
            gidx_v[pl.ds(g * L, L)] = e16 * N + s16
            seg_v[pl.ds(g * L, L)] = d16 * R + e16
            dch_v[pl.ds(g * L, L)] = d16
        cp_r = pltpu.async_copy(hw.at[gidx_v], row_v, gsem)
        cp_w = pltpu.async_copy(w_h.at[seg_v], wch_v, wsem)
        cp_r.wait()
        cp_w.wait()

        def group(g, inner):
            w16 = wch_v[pl.ds(g * L, L)]
            for e in range(L):
                wv = jnp.full((L,), w16[e], jnp.float32)
                eabs = g * L + e
                for j in range(D // L):
                    row_v[eabs, pl.ds(j * L, L)] = (
                        row_v[eabs, pl.ds(j * L, L)] * wv)
            return inner

        lax.fori_loop(0, CH // L, group, 0)
        pltpu.sync_copy(row_v, acc_sh.at[dch_v], add=True)
        return carry

    lax.fori_loop(0, NCH_AGG, chunk, 0)
    plsc.subcore_barrier()

    def flush(i, carry):
        pltpu.sync_copy(acc_sh.at[pl.ds(roff + i * CH, CH)], row_v)
        pltpu.sync_copy(row_v, out.at[c, pl.ds(roff + i * CH, CH)])
        return carry

    lax.fori_loop(0, ROWS_PT // CH, flush, 0)
    fo = pl.multiple_of(roff + ROWS_PT - _rem, 8)
    pltpu.sync_copy(acc_sh.at[pl.ds(fo, _rem)], row_v.at[pl.ds(0, _rem)])
    pltpu.sync_copy(row_v.at[pl.ds(0, _rem)], out.at[c, pl.ds(fo, _rem)])

    @pl.when(s == NS - 1)
    def _write_tail():
        pltpu.sync_copy(acc_sh.at[pl.ds(toff, ROWS_TAIL)],
                        row_v.at[pl.ds(0, ROWS_TAIL)])
        pltpu.sync_copy(row_v.at[pl.ds(0, ROWS_TAIL)],
                        out.at[c, pl.ds(toff, ROWS_TAIL)])


def _weights_kernel(cnt):
    def body(cnt_ref, w_ref):
        w_ref[...] = 1.0 / jnp.maximum(cnt_ref[...], 1.0)

    w = pl.pallas_call(
        body,
        out_shape=jax.ShapeDtypeStruct((2, NSEG // D, D), jnp.float32),
    )(cnt.reshape(2, NSEG // D, D))
    return w.reshape(2, NSEG)


_BN = 1000


def _dense1_kernel(x, Wm, bm, W1, root1, b1):
    def body(x_ref, wm_ref, bm_ref, w1_ref, rt_ref, b1_ref, hw_ref, hr_ref):
        h = jnp.maximum(
            jnp.dot(x_ref[...], wm_ref[...], preferred_element_type=jnp.float32)
            + bm_ref[...], 0.0)
        for r in range(R):
            hw_ref[r] = jnp.dot(h, w1_ref[r], preferred_element_type=jnp.float32)
        hr_ref[...] = jnp.dot(h, rt_ref[...],
                              preferred_element_type=jnp.float32) + b1_ref[...]

    return pl.pallas_call(
        body,
        grid=(N // _BN,),
        in_specs=[
            pl.BlockSpec((_BN, D), lambda i: (i, 0)),
            pl.BlockSpec((D, D), lambda i: (0, 0)),
            pl.BlockSpec((1, D), lambda i: (0, 0)),
            pl.BlockSpec((R, D, D), lambda i: (0, 0, 0)),
            pl.BlockSpec((D, D), lambda i: (0, 0)),
            pl.BlockSpec((1, D), lambda i: (0, 0)),
        ],
        out_specs=[
            pl.BlockSpec((R, _BN, D), lambda i: (0, i, 0)),
            pl.BlockSpec((_BN, D), lambda i: (i, 0)),
        ],
        out_shape=[
            jax.ShapeDtypeStruct((R, N, D), jnp.float32),
            jax.ShapeDtypeStruct((N, D), jnp.float32),
        ],
    )(x, Wm, bm.reshape(1, D), W1, root1, b1.reshape(1, D))


def _dense2_kernel(acc, hr_prev, W, root, b):
    def body(acc_ref, hp_ref, w_ref, rt_ref, b_ref, hw_ref, hr_ref):
        h = jnp.maximum(acc_ref[0] + acc_ref[1] + hp_ref[...], 0.0)
        for r in range(R):
            hw_ref[r] = jnp.dot(h, w_ref[r], preferred_element_type=jnp.float32)
        hr_ref[...] = jnp.dot(h, rt_ref[...],
                              preferred_element_type=jnp.float32) + b_ref[...]

    return pl.pallas_call(
        body,
        grid=(N // _BN,),
        in_specs=[
            pl.BlockSpec((NC, _BN, D), lambda i: (0, i, 0)),
            pl.BlockSpec((_BN, D), lambda i: (i, 0)),
            pl.BlockSpec((R, D, D), lambda i: (0, 0, 0)),
            pl.BlockSpec((D, D), lambda i: (0, 0)),
            pl.BlockSpec((1, D), lambda i: (0, 0)),
        ],
        out_specs=[
            pl.BlockSpec((R, _BN, D), lambda i: (0, i, 0)),
            pl.BlockSpec((_BN, D), lambda i: (i, 0)),
        ],
        out_shape=[
            jax.ShapeDtypeStruct((R, N, D), jnp.float32),
            jax.ShapeDtypeStruct((N, D), jnp.float32),
        ],
    )(acc, hr_prev, W, root, b.reshape(1, D))


def _final_kernel(acc, hr_prev, Wc, bc):
    def body(acc_ref, hp_ref, wc_ref, bc_ref, out_ref):
        h = jnp.maximum(acc_ref[0] + acc_ref[1] + hp_ref[...], 0.0)
        out_ref[...] = jnp.dot(h, wc_ref[...],
                               preferred_element_type=jnp.float32) + bc_ref[...]

    return pl.pallas_call(
        body,
        grid=(NUM_REPOS // _BN,),
        in_specs=[
            pl.BlockSpec((NC, _BN, D), lambda i: (0, i, 0)),
            pl.BlockSpec((_BN, D), lambda i: (i, 0)),
            pl.BlockSpec((D, D_OUT), lambda i: (0, 0)),
            pl.BlockSpec((1, D_OUT), lambda i: (0, 0)),
        ],
        out_specs=pl.BlockSpec((_BN, D_OUT), lambda i: (i, 0)),
        out_shape=jax.ShapeDtypeStruct((NUM_REPOS, D_OUT), jnp.float32),
    )(acc, hr_prev, Wc, bc.reshape(1, D_OUT))


def kernel(x, edge_index_ru, edge_index_ur, edge_type_ru, edge_type_ur,
           Wm, bm, W1, root1, b1, W2, root2, b2, Wc, bc):
    src1, dst1 = edge_index_ru[0], edge_index_ru[1]
    src2, dst2 = edge_index_ur[0], edge_index_ur[1]
    dst_cat = jnp.concatenate([dst1, dst2])
    et_cat = jnp.concatenate([edge_type_ru, edge_type_ur])

    cnt = _sc_counts(dst_cat, et_cat).reshape(2, NSEG)
    w = _weights_kernel(cnt)

    hw1, hr1 = _dense1_kernel(x, Wm, bm, W1, root1, b1)
    acc1 = _sc_agg(hw1.reshape(R * N, D), src1, dst1, edge_type_ru, w[0])
    hw2, hr2 = _dense2_kernel(acc1, hr1, W2, root2, b2)
    acc2 = _sc_agg(hw2.reshape(R * N, D), src2, dst2, edge_type_ur, w[1])
    return _final_kernel(acc2, hr2, Wc, bc)


# speedup vs baseline: 22.2562x; 1.0564x over previous
import functools

import jax
import jax.numpy as jnp
from jax import lax
from jax.experimental import pallas as pl
from jax.experimental.pallas import tpu as pltpu
from jax.experimental.pallas import tpu_sc as plsc

N = 10000
E = 320000
R = 8
D = 128
D_OUT = 16
NUM_REPOS = 5000
NSEG = N * R

NC = 2
NS = 16
L = 16

CH = 80
EPT_CNT = E // NS
EPT_AGG = E // (NC * NS)
NCH_CNT = EPT_CNT // CH
NCH_AGG = EPT_AGG // CH
SEG_PT = NSEG // NS
ROWS_PT = 624
ROWS_TAIL = N - ROWS_PT * NS

_SC_MESH = plsc.VectorSubcoreMesh(
    core_axis_name="c", subcore_axis_name="s", num_cores=NC, num_subcores=NS)


@functools.partial(
    pl.kernel,
    out_type=jax.ShapeDtypeStruct((NC, N, D), jnp.float32),
    mesh=_SC_MESH,
    scratch_types=[
        pltpu.VMEM((EPT_CNT,), jnp.int32),
        pltpu.VMEM((8, L), jnp.float32),
        pltpu.VMEM((CH, D), jnp.float32),
        pltpu.VMEM((CH,), jnp.int32),
        pltpu.VMEM_SHARED((N, D), jnp.float32),
    ],
)
def _sc_counts(seg_cat, eye_h, out, seg_sl, eye_v, onh_v, dch_v, cnt_sh):
    c = lax.axis_index("c")
    s = lax.axis_index("s")
    roff = pl.multiple_of(s * ROWS_PT, 8)
    toff = pl.multiple_of(ROWS_PT * NS, 8)
    pltpu.sync_copy(eye_h, eye_v)

    def zfill(i, carry):
        for j in range(D // L):
            onh_v[i, pl.ds(j * L, L)] = jnp.zeros((L,), jnp.float32)
        return carry

    lax.fori_loop(0, CH, zfill, 0)
    for k in range(ROWS_PT // CH):
        pltpu.sync_copy(onh_v, cnt_sh.at[pl.ds(roff + k * CH, CH)])
    _crem = ROWS_PT - (ROWS_PT // CH) * CH
    pltpu.sync_copy(onh_v.at[pl.ds(0, _crem)],
                    cnt_sh.at[pl.ds(roff + ROWS_PT - _crem, _crem)])

    @pl.when(s == NS - 1)
    def _zero_tail():
        pltpu.sync_copy(onh_v.at[pl.ds(0, ROWS_TAIL)],
                        cnt_sh.at[pl.ds(toff, ROWS_TAIL)])

    base = pl.multiple_of(c * E + s * EPT_CNT, 8)
    pltpu.sync_copy(seg_cat.at[pl.ds(base, EPT_CNT)], seg_sl)
    plsc.subcore_barrier()

    def chunk(i, carry):
        off = i * CH
        for g in range(CH // L):
            sg16 = seg_sl[pl.ds(off + g * L, L)]
            et16 = jnp.bitwise_and(sg16, R - 1)
            dch_v[pl.ds(g * L, L)] = jnp.right_shift(sg16, 3)
            for e in range(L):
                onh_v[g * L + e, pl.ds(0, L)] = eye_v[et16[e], pl.ds(0, L)]
        pltpu.sync_copy(onh_v, cnt_sh.at[dch_v], add=True)
        return carry

    lax.fori_loop(0, NCH_CNT, chunk, 0)
    plsc.subcore_barrier()

    def flush(i, carry):
        pltpu.sync_copy(cnt_sh.at[pl.ds(roff + i * CH, CH)], onh_v)
        pltpu.sync_copy(onh_v, out.at[c, pl.ds(roff + i * CH, CH)])
        return carry

    lax.fori_loop(0, ROWS_PT // CH, flush, 0)
    cfo = pl.multiple_of(roff + ROWS_PT - _crem, 8)
    pltpu.sync_copy(cnt_sh.at[pl.ds(cfo, _crem)], onh_v.at[pl.ds(0, _crem)])
    pltpu.sync_copy(onh_v.at[pl.ds(0, _crem)], out.at[c, pl.ds(cfo, _crem)])

    @pl.when(s == NS - 1)
    def _write_tail():
        pltpu.sync_copy(cnt_sh.at[pl.ds(toff, ROWS_TAIL)],
                        onh_v.at[pl.ds(0, ROWS_TAIL)])
        pltpu.sync_copy(onh_v.at[pl.ds(0, ROWS_TAIL)],
                        out.at[c, pl.ds(toff, ROWS_TAIL)])


@functools.partial(
    pl.kernel,
    out_type=jax.ShapeDtypeStruct((NC, N, D), jnp.float32),
    mesh=_SC_MESH,
    scratch_types=[
        pltpu.VMEM((EPT_AGG,), jnp.int32),
        pltpu.VMEM((EPT_AGG,), jnp.int32),
        pltpu.VMEM((CH,), jnp.int32),
        pltpu.VMEM((CH,), jnp.int32),
        pltpu.VMEM((CH,), jnp.int32),
        pltpu.VMEM((CH,), jnp.int32),
        pltpu.VMEM((CH,), jnp.int32),
        pltpu.VMEM((CH,), jnp.int32),
        pltpu.VMEM((CH,), jnp.float32),
        pltpu.VMEM((CH,), jnp.float32),
        pltpu.VMEM((CH, D), jnp.float32),
        pltpu.VMEM((CH, D), jnp.float32),
        pltpu.VMEM_SHARED((N, D), jnp.float32),
        pltpu.SemaphoreType.DMA,
        pltpu.SemaphoreType.DMA,
        pltpu.SemaphoreType.DMA,
        pltpu.SemaphoreType.DMA,
    ],
)
def _sc_agg(hw, gidx_h, seg_h, w_h, out,
            gidx_sl, seg_sl, gidx_a, gidx_b, seg_a, seg_b, dch_a, dch_b,
            wch_a, wch_b, row_a, row_b,
            acc_sh, gsem0, gsem1, wsem0, wsem1):
    c = lax.axis_index("c")
    s = lax.axis_index("s")
    base = pl.multiple_of((c * NS + s) * EPT_AGG, 8)
    roff = pl.multiple_of(s * ROWS_PT, 8)
    toff = pl.multiple_of(ROWS_PT * NS, 8)

    rv0 = row_a
    gsems = (gsem0, gsem1)
    wsems = (wsem0, wsem1)
    gidxs = (gidx_a, gidx_b)
    segs = (seg_a, seg_b)
    dchs = (dch_a, dch_b)
    wchs = (wch_a, wch_b)
    rows = (row_a, row_b)

    def zfill(i, carry):
        for j in range(D // L):
            row_a[i, pl.ds(j * L, L)] = jnp.zeros((L,), jnp.float32)
        return carry

    lax.fori_loop(0, CH, zfill, 0)
    for k in range(ROWS_PT // CH):
        pltpu.sync_copy(rv0, acc_sh.at[pl.ds(roff + k * CH, CH)])
    _rem = ROWS_PT - (ROWS_PT // CH) * CH
    pltpu.sync_copy(rv0.at[pl.ds(0, _rem)],
                    acc_sh.at[pl.ds(roff + ROWS_PT - _rem, _rem)])

    @pl.when(s == NS - 1)
    def _zero_tail():
        pltpu.sync_copy(rv0.at[pl.ds(0, ROWS_TAIL)],
                        acc_sh.at[pl.ds(toff, ROWS_TAIL)])

    pltpu.sync_copy(gidx_h.at[pl.ds(base, EPT_AGG)], gidx_sl)
    pltpu.sync_copy(seg_h.at[pl.ds(base, EPT_AGG)], seg_sl)
    plsc.subcore_barrier()

    def stage(k, b):
        off = k * CH
        for g in range(CH // L):
            sg16 = seg_sl[pl.ds(off + g * L, L)]
            gidxs[b][pl.ds(g * L, L)] = gidx_sl[pl.ds(off + g * L, L)]
            segs[b][pl.ds(g * L, L)] = sg16
            dchs[b][pl.ds(g * L, L)] = jnp.right_shift(sg16, 3)
        pltpu.async_copy(hw.at[gidxs[b]], rows[b], gsems[b])
        pltpu.async_copy(w_h.at[segs[b]], wchs[b], wsems[b])

    def process(b):
        pltpu.make_async_copy(hw.at[gidxs[b]], rows[b], gsems[b]).wait()
        pltpu.make_async_copy(w_h.at[segs[b]], wchs[b], wsems[b]).wait()

        def group(g, inner):
            w16 = wchs[b][pl.ds(g * L, L)]
            for e in range(L):
                wv = jnp.full((L,), w16[e], jnp.float32)
                eabs = g * L + e
                for j in range(D // L):
                    rows[b][eabs, pl.ds(j * L, L)] = (
                        rows[b][eabs, pl.ds(j * L, L)] * wv)
            return inner

        lax.fori_loop(0, CH // L, group, 0)
        pltpu.sync_copy(rows[b], acc_sh.at[dchs[b]], add=True)

    stage(0, 0)

    def pair(p, carry):
        k0 = 2 * p
        stage(k0 + 1, 1)
        process(0)
        stage(k0 + 2, 0)
        process(1)
        return carry

    lax.fori_loop(0, (NCH_AGG - 1) // 2, pair, 0)
    process(0)

    plsc.subcore_barrier()

    def flush(i, carry):
        pltpu.sync_copy(acc_sh.at[pl.ds(roff + i * CH, CH)], rv0)
        pltpu.sync_copy(rv0, out.at[c, pl.ds(roff + i * CH, CH)])
        return carry

    lax.fori_loop(0, ROWS_PT // CH, flush, 0)
    fo = pl.multiple_of(roff + ROWS_PT - _rem, 8)
    pltpu.sync_copy(acc_sh.at[pl.ds(fo, _rem)], rv0.at[pl.ds(0, _rem)])
    pltpu.sync_copy(rv0.at[pl.ds(0, _rem)], out.at[c, pl.ds(fo, _rem)])

    @pl.when(s == NS - 1)
    def _write_tail():
        pltpu.sync_copy(acc_sh.at[pl.ds(toff, ROWS_TAIL)],
                        rv0.at[pl.ds(0, ROWS_TAIL)])
        pltpu.sync_copy(rv0.at[pl.ds(0, ROWS_TAIL)],
                        out.at[c, pl.ds(toff, ROWS_TAIL)])


_EROWS = 2 * E // D


def _edge_prep_kernel(src_cat, et_cat, dst_cat):
    def body(src_ref, et_ref, dst_ref, gidx_ref, seg_ref):
        e = et_ref[...]
        gidx_ref[...] = e * N + src_ref[...]
        seg_ref[...] = dst_ref[...] * R + e

    gidx, seg = pl.pallas_call(
        body,
        grid=(5,),
        in_specs=[pl.BlockSpec((_EROWS // 5, D), lambda i: (i, 0))] * 3,
        out_specs=[pl.BlockSpec((_EROWS // 5, D), lambda i: (i, 0))] * 2,
        out_shape=[jax.ShapeDtypeStruct((_EROWS, D), jnp.int32)] * 2,
    )(src_cat.reshape(_EROWS, D), et_cat.reshape(_EROWS, D),
      dst_cat.reshape(_EROWS, D))
    return gidx.reshape(2 * E), seg.reshape(2 * E)


def _weights_kernel(cnt):
    def body(cnt_ref, w_ref):
        w_ref[...] = 1.0 / jnp.maximum(cnt_ref[..., :R], 1.0)

    w = pl.pallas_call(
        body,
        grid=(10,),
        in_specs=[pl.BlockSpec((2, N // 10, D), lambda i: (0, i, 0))],
        out_specs=pl.BlockSpec((2, N // 10, R), lambda i: (0, i, 0)),
        out_shape=jax.ShapeDtypeStruct((2, N, R), jnp.float32),
    )(cnt)
    return w.reshape(2, NSEG)


_BN = 1000


def _dense1_kernel(x, Wm, bm, W1, root1, b1):
    def body(x_ref, wm_ref, bm_ref, w1_ref, rt_ref, b1_ref, hw_ref, hr_ref):
        h = jnp.maximum(
            jnp.dot(x_ref[...], wm_ref[...], preferred_element_type=jnp.float32)
            + bm_ref[...], 0.0)
        for r in range(R):
            hw_ref[r] = jnp.dot(h, w1_ref[r], preferred_element_type=jnp.float32)
        hr_ref[...] = jnp.dot(h, rt_ref[...],
                              preferred_element_type=jnp.float32) + b1_ref[...]

    return pl.pallas_call(
        body,
        grid=(N // _BN,),
        in_specs=[
            pl.BlockSpec((_BN, D), lambda i: (i, 0)),
            pl.BlockSpec((D, D), lambda i: (0, 0)),
            pl.BlockSpec((1, D), lambda i: (0, 0)),
            pl.BlockSpec((R, D, D), lambda i: (0, 0, 0)),
            pl.BlockSpec((D, D), lambda i: (0, 0)),
            pl.BlockSpec((1, D), lambda i: (0, 0)),
        ],
        out_specs=[
            pl.BlockSpec((R, _BN, D), lambda i: (0, i, 0)),
            pl.BlockSpec((_BN, D), lambda i: (i, 0)),
        ],
        out_shape=[
            jax.ShapeDtypeStruct((R, N, D), jnp.float32),
            jax.ShapeDtypeStruct((N, D), jnp.float32),
        ],
    )(x, Wm, bm.reshape(1, D), W1, root1, b1.reshape(1, D))


def _dense2_kernel(acc, hr_prev, W, root, b):
    def body(acc_ref, hp_ref, w_ref, rt_ref, b_ref, hw_ref, hr_ref):
        h = jnp.maximum(acc_ref[0] + acc_ref[1] + hp_ref[...], 0.0)
        for r in range(R):
            hw_ref[r] = jnp.dot(h, w_ref[r], preferred_element_type=jnp.float32)
        hr_ref[...] = jnp.dot(h, rt_ref[...],
                              preferred_element_type=jnp.float32) + b_ref[...]

    return pl.pallas_call(
        body,
        grid=(N // _BN,),
        in_specs=[
            pl.BlockSpec((NC, _BN, D), lambda i: (0, i, 0)),
            pl.BlockSpec((_BN, D), lambda i: (i, 0)),
            pl.BlockSpec((R, D, D), lambda i: (0, 0, 0)),
            pl.BlockSpec((D, D), lambda i: (0, 0)),
            pl.BlockSpec((1, D), lambda i: (0, 0)),
        ],
        out_specs=[
            pl.BlockSpec((R, _BN, D), lambda i: (0, i, 0)),
            pl.BlockSpec((_BN, D), lambda i: (i, 0)),
        ],
        out_shape=[
            jax.ShapeDtypeStruct((R, N, D), jnp.float32),
            jax.ShapeDtypeStruct((N, D), jnp.float32),
        ],
    )(acc, hr_prev, W, root, b.reshape(1, D))


def _final_kernel(acc, hr_prev, Wc, bc):
    def body(acc_ref, hp_ref, wc_ref, bc_ref, out_ref):
        h = jnp.maximum(acc_ref[0] + acc_ref[1] + hp_ref[...], 0.0)
        out_ref[...] = jnp.dot(h, wc_ref[...],
                               preferred_element_type=jnp.float32) + bc_ref[...]

    return pl.pallas_call(
        body,
        grid=(NUM_REPOS // _BN,),
        in_specs=[
            pl.BlockSpec((NC, _BN, D), lambda i: (0, i, 0)),
            pl.BlockSpec((_BN, D), lambda i: (i, 0)),
            pl.BlockSpec((D, D_OUT), lambda i: (0, 0)),
            pl.BlockSpec((1, D_OUT), lambda i: (0, 0)),
        ],
        out_specs=pl.BlockSpec((_BN, D_OUT), lambda i: (i, 0)),
        out_shape=jax.ShapeDtypeStruct((NUM_REPOS, D_OUT), jnp.float32),
    )(acc, hr_prev, Wc, bc.reshape(1, D_OUT))


def kernel(x, edge_index_ru, edge_index_ur, edge_type_ru, edge_type_ur,
           Wm, bm, W1, root1, b1, W2, root2, b2, Wc, bc):
    src1, dst1 = edge_index_ru[0], edge_index_ru[1]
    src2, dst2 = edge_index_ur[0], edge_index_ur[1]
    src_cat = jnp.concatenate([src1, src2])
    dst_cat = jnp.concatenate([dst1, dst2])
    et_cat = jnp.concatenate([edge_type_ru, edge_type_ur])

    gidx_cat, seg_cat = _edge_prep_kernel(src_cat, et_cat, dst_cat)
    cnt = _sc_counts(seg_cat, jnp.eye(8, L, dtype=jnp.float32))
    w = _weights_kernel(cnt)

    hw1, hr1 = _dense1_kernel(x, Wm, bm, W1, root1, b1)
    acc1 = _sc_agg(hw1.reshape(R * N, D), gidx_cat[:E], seg_cat[:E], w[0])
    hw2, hr2 = _dense2_kernel(acc1, hr1, W2, root2, b2)
    acc2 = _sc_agg(hw2.reshape(R * N, D), gidx_cat[E:], seg_cat[E:], w[1])
    return _final_kernel(acc2, hr2, Wc, bc)


# speedup vs baseline: 25.8001x; 1.1592x over previous
import functools

import jax
import jax.numpy as jnp
from jax import lax
from jax.experimental import pallas as pl
from jax.experimental.pallas import tpu as pltpu
from jax.experimental.pallas import tpu_sc as plsc

N = 10000
E = 320000
R = 8
D = 128
D_OUT = 16
NUM_REPOS = 5000
NSEG = N * R

NC = 2
NS = 16
L = 16

CH = 80
EPT_CNT = E // NS
EPT_AGG = E // (NC * NS)
NCH_CNT = EPT_CNT // CH
NCH_AGG = EPT_AGG // CH
SEG_PT = NSEG // NS
ROWS_PT = 624
ROWS_TAIL = N - ROWS_PT * NS

_SC_MESH = plsc.VectorSubcoreMesh(
    core_axis_name="c", subcore_axis_name="s", num_cores=NC, num_subcores=NS)


@functools.partial(
    pl.kernel,
    out_type=jax.ShapeDtypeStruct((NC, N, D), jnp.float32),
    mesh=_SC_MESH,
    scratch_types=[
        pltpu.VMEM((EPT_CNT,), jnp.int32),
        pltpu.VMEM((CH, D), jnp.float32),
        pltpu.VMEM((CH, D), jnp.float32),
        pltpu.VMEM((CH,), jnp.int32),
        pltpu.VMEM((CH,), jnp.int32),
        pltpu.VMEM_SHARED((N, D), jnp.float32),
        pltpu.SemaphoreType.DMA,
        pltpu.SemaphoreType.DMA,
    ],
)
def _sc_counts(seg_cat, out, seg_sl, onh_a, onh_b, dch_a, dch_b, cnt_sh,
               ssem0, ssem1):
    c = lax.axis_index("c")
    s = lax.axis_index("s")
    onhs = (onh_a, onh_b)
    dchs = (dch_a, dch_b)
    ssems = (ssem0, ssem1)
    lanes = lax.iota(jnp.int32, L)
    roff = pl.multiple_of(s * ROWS_PT, 8)
    toff = pl.multiple_of(ROWS_PT * NS, 8)

    def zfill(i, carry):
        for j in range(D // L):
            onh_a[i, pl.ds(j * L, L)] = jnp.zeros((L,), jnp.float32)
            onh_b[i, pl.ds(j * L, L)] = jnp.zeros((L,), jnp.float32)
        return carry

    lax.fori_loop(0, CH, zfill, 0)
    for k in range(ROWS_PT // CH):
        pltpu.sync_copy(onh_a, cnt_sh.at[pl.ds(roff + k * CH, CH)])
    _crem = ROWS_PT - (ROWS_PT // CH) * CH
    pltpu.sync_copy(onh_a.at[pl.ds(0, _crem)],
                    cnt_sh.at[pl.ds(roff + ROWS_PT - _crem, _crem)])

    @pl.when(s == NS - 1)
    def _zero_tail():
        pltpu.sync_copy(onh_a.at[pl.ds(0, ROWS_TAIL)],
                        cnt_sh.at[pl.ds(toff, ROWS_TAIL)])

    base = pl.multiple_of(c * E + s * EPT_CNT, 8)
    pltpu.sync_copy(seg_cat.at[pl.ds(base, EPT_CNT)], seg_sl)
    plsc.subcore_barrier()

    def cpair(p, carry):
        for b in range(2):
            k = 2 * p + b

            @pl.when(p >= 1)
            def _wait_prev():
                pltpu.make_async_copy(onhs[b], cnt_sh.at[dchs[b]],
                                      ssems[b]).wait()

            off = k * CH
            for g in range(CH // L):
                sg16 = seg_sl[pl.ds(off + g * L, L)]
                et16 = jnp.bitwise_and(sg16, R - 1)
                dchs[b][pl.ds(g * L, L)] = jnp.right_shift(sg16, 3)
                for e in range(L):
                    onhs[b][g * L + e, pl.ds(0, L)] = jnp.where(
                        lanes == et16[e], 1.0, 0.0).astype(jnp.float32)
            pltpu.async_copy(onhs[b], cnt_sh.at[dchs[b]], ssems[b], add=True)
        return carry

    lax.fori_loop(0, NCH_CNT // 2, cpair, 0)
    for b in range(2):
        pltpu.make_async_copy(onhs[b], cnt_sh.at[dchs[b]], ssems[b]).wait()
    plsc.subcore_barrier()

    def flush(i, carry):
        pltpu.sync_copy(cnt_sh.at[pl.ds(roff + i * CH, CH)], onh_a)
        pltpu.sync_copy(onh_a, out.at[c, pl.ds(roff + i * CH, CH)])
        return carry

    lax.fori_loop(0, ROWS_PT // CH, flush, 0)
    cfo = pl.multiple_of(roff + ROWS_PT - _crem, 8)
    pltpu.sync_copy(cnt_sh.at[pl.ds(cfo, _crem)], onh_a.at[pl.ds(0, _crem)])
    pltpu.sync_copy(onh_a.at[pl.ds(0, _crem)], out.at[c, pl.ds(cfo, _crem)])

    @pl.when(s == NS - 1)
    def _write_tail():
        pltpu.sync_copy(cnt_sh.at[pl.ds(toff, ROWS_TAIL)],
                        onh_a.at[pl.ds(0, ROWS_TAIL)])
        pltpu.sync_copy(onh_a.at[pl.ds(0, ROWS_TAIL)],
                        out.at[c, pl.ds(toff, ROWS_TAIL)])


@functools.partial(
    pl.kernel,
    out_type=jax.ShapeDtypeStruct((NC, N, D), jnp.float32),
    mesh=_SC_MESH,
    scratch_types=[
        pltpu.VMEM((EPT_AGG,), jnp.int32),
        pltpu.VMEM((EPT_AGG,), jnp.int32),
        pltpu.VMEM((CH,), jnp.int32),
        pltpu.VMEM((CH,), jnp.int32),
        pltpu.VMEM((CH,), jnp.int32),
        pltpu.VMEM((CH,), jnp.int32),
        pltpu.VMEM((CH,), jnp.int32),
        pltpu.VMEM((CH,), jnp.int32),
        pltpu.VMEM((CH,), jnp.float32),
        pltpu.VMEM((CH,), jnp.float32),
        pltpu.VMEM((CH, D), jnp.float32),
        pltpu.VMEM((CH, D), jnp.float32),
        pltpu.VMEM_SHARED((N, D), jnp.float32),
        pltpu.SemaphoreType.DMA,
        pltpu.SemaphoreType.DMA,
        pltpu.SemaphoreType.DMA,
        pltpu.SemaphoreType.DMA,
    ],
)
def _sc_agg(hw, gidx_h, seg_h, w_h, out,
            gidx_sl, seg_sl, gidx_a, gidx_b, seg_a, seg_b, dch_a, dch_b,
            wch_a, wch_b, row_a, row_b,
            acc_sh, gsem0, gsem1, wsem0, wsem1):
    c = lax.axis_index("c")
    s = lax.axis_index("s")
    base = pl.multiple_of((c * NS + s) * EPT_AGG, 8)
    roff = pl.multiple_of(s * ROWS_PT, 8)
    toff = pl.multiple_of(ROWS_PT * NS, 8)

    rv0 = row_a
    gsems = (gsem0, gsem1)
    wsems = (wsem0, wsem1)
    gidxs = (gidx_a, gidx_b)
    segs = (seg_a, seg_b)
    dchs = (dch_a, dch_b)
    wchs = (wch_a, wch_b)
    rows = (row_a, row_b)

    def zfill(i, carry):
        for j in range(D // L):
            row_a[i, pl.ds(j * L, L)] = jnp.zeros((L,), jnp.float32)
        return carry

    lax.fori_loop(0, CH, zfill, 0)
    for k in range(ROWS_PT // CH):
        pltpu.sync_copy(rv0, acc_sh.at[pl.ds(roff + k * CH, CH)])
    _rem = ROWS_PT - (ROWS_PT // CH) * CH
    pltpu.sync_copy(rv0.at[pl.ds(0, _rem)],
                    acc_sh.at[pl.ds(roff + ROWS_PT - _rem, _rem)])

    @pl.when(s == NS - 1)
    def _zero_tail():
        pltpu.sync_copy(rv0.at[pl.ds(0, ROWS_TAIL)],
                        acc_sh.at[pl.ds(toff, ROWS_TAIL)])

    pltpu.sync_copy(gidx_h.at[pl.ds(base, EPT_AGG)], gidx_sl)
    pltpu.sync_copy(seg_h.at[pl.ds(base, EPT_AGG)], seg_sl)
    plsc.subcore_barrier()

    def stage(k, b):
        off = k * CH
        for g in range(CH // L):
            sg16 = seg_sl[pl.ds(off + g * L, L)]
            gidxs[b][pl.ds(g * L, L)] = gidx_sl[pl.ds(off + g * L, L)]
            segs[b][pl.ds(g * L, L)] = sg16
            dchs[b][pl.ds(g * L, L)] = jnp.right_shift(sg16, 3)
        pltpu.async_copy(hw.at[gidxs[b]], rows[b], gsems[b])
        pltpu.async_copy(w_h.at[segs[b]], wchs[b], wsems[b])

    def process(b):
        pltpu.make_async_copy(hw.at[gidxs[b]], rows[b], gsems[b]).wait()
        pltpu.make_async_copy(w_h.at[segs[b]], wchs[b], wsems[b]).wait()

        def group(g, inner):
            w16 = wchs[b][pl.ds(g * L, L)]
            for e in range(L):
                wv = jnp.full((L,), w16[e], jnp.float32)
                eabs = g * L + e
                for j in range(D // L):
                    rows[b][eabs, pl.ds(j * L, L)] = (
                        rows[b][eabs, pl.ds(j * L, L)] * wv)
            return inner

        lax.fori_loop(0, CH // L, group, 0)
        pltpu.sync_copy(rows[b], acc_sh.at[dchs[b]], add=True)

    stage(0, 0)

    def pair(p, carry):
        k0 = 2 * p
        stage(k0 + 1, 1)
        process(0)
        stage(k0 + 2, 0)
        process(1)
        return carry

    lax.fori_loop(0, (NCH_AGG - 1) // 2, pair, 0)
    process(0)

    plsc.subcore_barrier()

    def flush(i, carry):
        pltpu.sync_copy(acc_sh.at[pl.ds(roff + i * CH, CH)], rv0)
        pltpu.sync_copy(rv0, out.at[c, pl.ds(roff + i * CH, CH)])
        return carry

    lax.fori_loop(0, ROWS_PT // CH, flush, 0)
    fo = pl.multiple_of(roff + ROWS_PT - _rem, 8)
    pltpu.sync_copy(acc_sh.at[pl.ds(fo, _rem)], rv0.at[pl.ds(0, _rem)])
    pltpu.sync_copy(rv0.at[pl.ds(0, _rem)], out.at[c, pl.ds(fo, _rem)])

    @pl.when(s == NS - 1)
    def _write_tail():
        pltpu.sync_copy(acc_sh.at[pl.ds(toff, ROWS_TAIL)],
                        rv0.at[pl.ds(0, ROWS_TAIL)])
        pltpu.sync_copy(rv0.at[pl.ds(0, ROWS_TAIL)],
                        out.at[c, pl.ds(toff, ROWS_TAIL)])


_EROWS = 2 * E // D


def _edge_prep_kernel(src_cat, et_cat, dst_cat):
    def body(src_ref, et_ref, dst_ref, gidx_ref, seg_ref):
        e = et_ref[...]
        gidx_ref[...] = e * N + src_ref[...]
        seg_ref[...] = dst_ref[...] * R + e

    gidx, seg = pl.pallas_call(
        body,
        grid=(5,),
        in_specs=[pl.BlockSpec((_EROWS // 5, D), lambda i: (i, 0))] * 3,
        out_specs=[pl.BlockSpec((_EROWS // 5, D), lambda i: (i, 0))] * 2,
        out_shape=[jax.ShapeDtypeStruct((_EROWS, D), jnp.int32)] * 2,
    )(src_cat.reshape(_EROWS, D), et_cat.reshape(_EROWS, D),
      dst_cat.reshape(_EROWS, D))
    return gidx.reshape(2 * E), seg.reshape(2 * E)


def _weights_kernel(cnt):
    def body(cnt_ref, w_ref):
        w_ref[...] = 1.0 / jnp.maximum(cnt_ref[..., :R], 1.0)

    w = pl.pallas_call(
        body,
        grid=(10,),
        in_specs=[pl.BlockSpec((2, N // 10, D), lambda i: (0, i, 0))],
        out_specs=pl.BlockSpec((2, N // 10, R), lambda i: (0, i, 0)),
        out_shape=jax.ShapeDtypeStruct((2, N, R), jnp.float32),
    )(cnt)
    return w.reshape(2, NSEG)


_BN = 1000


def _dense1_kernel(x, Wm, bm, W1, root1, b1):
    def body(x_ref, wm_ref, bm_ref, w1_ref, rt_ref, b1_ref, hw_ref, hr_ref):
        h = jnp.maximum(
            jnp.dot(x_ref[...], wm_ref[...], preferred_element_type=jnp.float32)
            + bm_ref[...], 0.0)
        for r in range(R):
            hw_ref[r] = jnp.dot(h, w1_ref[r], preferred_element_type=jnp.float32)
        hr_ref[...] = jnp.dot(h, rt_ref[...],
                              preferred_element_type=jnp.float32) + b1_ref[...]

    return pl.pallas_call(
        body,
        grid=(N // _BN,),
        in_specs=[
            pl.BlockSpec((_BN, D), lambda i: (i, 0)),
            pl.BlockSpec((D, D), lambda i: (0, 0)),
            pl.BlockSpec((1, D), lambda i: (0, 0)),
            pl.BlockSpec((R, D, D), lambda i: (0, 0, 0)),
            pl.BlockSpec((D, D), lambda i: (0, 0)),
            pl.BlockSpec((1, D), lambda i: (0, 0)),
        ],
        out_specs=[
            pl.BlockSpec((R, _BN, D), lambda i: (0, i, 0)),
            pl.BlockSpec((_BN, D), lambda i: (i, 0)),
        ],
        out_shape=[
            jax.ShapeDtypeStruct((R, N, D), jnp.float32),
            jax.ShapeDtypeStruct((N, D), jnp.float32),
        ],
    )(x, Wm, bm.reshape(1, D), W1, root1, b1.reshape(1, D))


def _dense2_kernel(acc, hr_prev, W, root, b):
    def body(acc_ref, hp_ref, w_ref, rt_ref, b_ref, hw_ref, hr_ref):
        h = jnp.maximum(acc_ref[0] + acc_ref[1] + hp_ref[...], 0.0)
        for r in range(R):
            hw_ref[r] = jnp.dot(h, w_ref[r], preferred_element_type=jnp.float32)
        hr_ref[...] = jnp.dot(h, rt_ref[...],
                              preferred_element_type=jnp.float32) + b_ref[...]

    return pl.pallas_call(
        body,
        grid=(N // _BN,),
        in_specs=[
            pl.BlockSpec((NC, _BN, D), lambda i: (0, i, 0)),
            pl.BlockSpec((_BN, D), lambda i: (i, 0)),
            pl.BlockSpec((R, D, D), lambda i: (0, 0, 0)),
            pl.BlockSpec((D, D), lambda i: (0, 0)),
            pl.BlockSpec((1, D), lambda i: (0, 0)),
        ],
        out_specs=[
            pl.BlockSpec((R, _BN, D), lambda i: (0, i, 0)),
            pl.BlockSpec((_BN, D), lambda i: (i, 0)),
        ],
        out_shape=[
            jax.ShapeDtypeStruct((R, N, D), jnp.float32),
            jax.ShapeDtypeStruct((N, D), jnp.float32),
        ],
    )(acc, hr_prev, W, root, b.reshape(1, D))


def _final_kernel(acc, hr_prev, Wc, bc):
    def body(acc_ref, hp_ref, wc_ref, bc_ref, out_ref):
        h = jnp.maximum(acc_ref[0] + acc_ref[1] + hp_ref[...], 0.0)
        out_ref[...] = jnp.dot(h, wc_ref[...],
                               preferred_element_type=jnp.float32) + bc_ref[...]

    return pl.pallas_call(
        body,
        grid=(NUM_REPOS // _BN,),
        in_specs=[
            pl.BlockSpec((NC, _BN, D), lambda i: (0, i, 0)),
            pl.BlockSpec((_BN, D), lambda i: (i, 0)),
            pl.BlockSpec((D, D_OUT), lambda i: (0, 0)),
            pl.BlockSpec((1, D_OUT), lambda i: (0, 0)),
        ],
        out_specs=pl.BlockSpec((_BN, D_OUT), lambda i: (i, 0)),
        out_shape=jax.ShapeDtypeStruct((NUM_REPOS, D_OUT), jnp.float32),
    )(acc, hr_prev, Wc, bc.reshape(1, D_OUT))


def kernel(x, edge_index_ru, edge_index_ur, edge_type_ru, edge_type_ur,
           Wm, bm, W1, root1, b1, W2, root2, b2, Wc, bc):
    src1, dst1 = edge_index_ru[0], edge_index_ru[1]
    src2, dst2 = edge_index_ur[0], edge_index_ur[1]
    src_cat = jnp.concatenate([src1, src2])
    dst_cat = jnp.concatenate([dst1, dst2])
    et_cat = jnp.concatenate([edge_type_ru, edge_type_ur])

    gidx_cat, seg_cat = _edge_prep_kernel(src_cat, et_cat, dst_cat)
    cnt = _sc_counts(seg_cat)
    w = _weights_kernel(cnt)

    hw1, hr1 = _dense1_kernel(x, Wm, bm, W1, root1, b1)
    acc1 = _sc_agg(hw1.reshape(R * N, D), gidx_cat[:E], seg_cat[:E], w[0])
    hw2, hr2 = _dense2_kernel(acc1, hr1, W2, root2, b2)
    acc2 = _sc_agg(hw2.reshape(R * N, D), gidx_cat[E:], seg_cat[E:], w[1])
    return _final_kernel(acc2, hr2, Wc, bc)
